# baseline (speedup 1.0000x reference)
import functools

import jax
import jax.numpy as jnp
from jax import lax
from jax.experimental import pallas as pl
from jax.experimental.pallas import tpu as pltpu
from jax.experimental.pallas import tpu_sc as plsc

N = 10000
E = 320000
D = 128

NC = 2
NS = 16
NW = NC * NS
EPW = E // NW
K = 128
CHUNKS = EPW // K
TK = EPW - CHUNKS * K
NB = 2
NI = 4
GROUPS = CHUNKS // NI
RPT = 624
TAIL0 = NS * RPT
ZR = 40

_mesh = plsc.VectorSubcoreMesh(
    core_axis_name="c", subcore_axis_name="s", num_cores=NC, num_subcores=NS)


def _zero_fill(zbuf, rows, width):
    def row(i, _):
        for j in range(width // 16):
            zbuf[i, pl.ds(j * 16, 16)] = jnp.zeros((16,), jnp.float32)
        return 0
    lax.fori_loop(0, rows, row, 0)


def _zero_shared(sh, zbuf, sid):
    base = sid * RPT
    def z(i, _):
        pltpu.sync_copy(zbuf, sh.at[pl.ds(base + i * ZR, ZR)])
        return 0
    lax.fori_loop(0, 640 // ZR, z, 0)


def _writeback(src_sh, dst_hbm, sid):
    r0 = sid * RPT
    pltpu.sync_copy(src_sh.at[pl.ds(r0, RPT)], dst_hbm.at[pl.ds(r0, RPT)])
    @pl.when(sid == NS - 1)
    def _():
        pltpu.sync_copy(src_sh.at[pl.ds(TAIL0, N - TAIL0)],
                        dst_hbm.at[pl.ds(TAIL0, N - TAIL0)])


@functools.partial(
    pl.kernel,
    out_type=(jax.ShapeDtypeStruct((N, D), jnp.float32),
              jax.ShapeDtypeStruct((N, D), jnp.float32)),
    mesh=_mesh,
    scratch_types=(
        [pltpu.VMEM_SHARED((N, D), jnp.float32)]
        + [pltpu.VMEM((K,), jnp.int32)] * NI
        + [pltpu.VMEM((K,), jnp.int32)] * NI
        + [pltpu.VMEM((TK,), jnp.int32)] * 2
        + [pltpu.VMEM((K, D), jnp.float32)] * NB
        + [pltpu.VMEM((TK, D), jnp.float32)]
        + [pltpu.VMEM((ZR, D), jnp.float32)]
        + [pltpu.SemaphoreType.DMA] * (2 * NB + NI)
    ),
)
def _sc_agg(x_hbm, src_hbm, dst_hbm, acc0_hbm, acc1_hbm, *s):
    acc_sh = s[0]
    sidx = s[1:1 + NI]
    didx = s[1 + NI:1 + 2 * NI]
    sidxt, didxt = s[1 + 2 * NI:3 + 2 * NI]
    rows = s[3 + 2 * NI:3 + 2 * NI + NB]
    rowst = s[3 + 2 * NI + NB]
    zbuf = s[4 + 2 * NI + NB]
    semg = s[5 + 2 * NI + NB:5 + 2 * NI + 2 * NB]
    sems = s[5 + 2 * NI + 2 * NB:5 + 2 * NI + 3 * NB]
    semi = s[5 + 2 * NI + 3 * NB:5 + 3 * NI + 3 * NB]
    cid = lax.axis_index("c")
    sid = lax.axis_index("s")
    wid = sid * NC + cid

    _zero_fill(zbuf, ZR, D)
    _zero_shared(acc_sh, zbuf, sid)
    plsc.subcore_barrier()

    ebase = wid * EPW

    def fire_idx(c, j):
        pltpu.async_copy(src_hbm.at[pl.ds(ebase + c * K, K)], sidx[j],
                         semi[j])
        pltpu.async_copy(dst_hbm.at[pl.ds(ebase + c * K, K)], didx[j],
                         semi[j])

    def wait_idx(j):
        pltpu.make_async_copy(src_hbm.at[pl.ds(ebase, K)], sidx[j],
                              semi[j]).wait()
        pltpu.make_async_copy(dst_hbm.at[pl.ds(ebase, K)], didx[j],
                              semi[j]).wait()

    def fire_gather(b, j):
        pltpu.async_copy(x_hbm.at[sidx[j]], rows[b], semg[b])

    def wait_gather(b, j):
        pltpu.make_async_copy(x_hbm.at[sidx[j]], rows[b], semg[b]).wait()

    def fire_scatter(b, j):
        pltpu.async_copy(rows[b], acc_sh.at[didx[j]], sems[b], add=True)

    def wait_scatter(b, j):
        pltpu.make_async_copy(rows[b], acc_sh.at[didx[j]], sems[b]).wait()

    for j in range(NI):
        fire_idx(j, j)
    wait_idx(0)
    fire_gather(0, 0)
    wait_idx(1)
    fire_gather(1, 1)

    def group(g, _):
        c0 = 4 * g
        wait_gather(0, 0)
        fire_scatter(0, 0)
        wait_gather(1, 1)
        fire_scatter(1, 1)
        wait_scatter(0, 0)
        wait_idx(2)
        fire_gather(0, 2)
        @pl.when(c0 + 4 < CHUNKS)
        def _():
            fire_idx(c0 + 4, 0)
        wait_scatter(1, 1)
        wait_idx(3)
        fire_gather(1, 3)
        @pl.when(c0 + 5 < CHUNKS)
        def _():
            fire_idx(c0 + 5, 1)
        wait_gather(0, 2)
        fire_scatter(0, 2)
        wait_gather(1, 3)
        fire_scatter(1, 3)
        wait_scatter(0, 2)
        @pl.when(c0 + 4 < CHUNKS)
        def _():
            wait_idx(0)
            fire_gather(0, 0)
        @pl.when(c0 + 6 < CHUNKS)
        def _():
            fire_idx(c0 + 6, 2)
        wait_scatter(1, 3)
        @pl.when(c0 + 5 < CHUNKS)
        def _():
            wait_idx(1)
            fire_gather(1, 1)
        @pl.when(c0 + 7 < CHUNKS)
        def _():
            fire_idx(c0 + 7, 3)
        return 0
    lax.fori_loop(0, GROUPS, group, 0)

    wait_gather(0, 0)
    fire_scatter(0, 0)
    wait_gather(1, 1)
    fire_scatter(1, 1)
    wait_scatter(0, 0)
    wait_scatter(1, 1)

    tbase = ebase + CHUNKS * K
    pltpu.sync_copy(src_hbm.at[pl.ds(tbase, TK)], sidxt)
    pltpu.async_copy(x_hbm.at[sidxt], rowst, semg[0]).wait()
    pltpu.sync_copy(dst_hbm.at[pl.ds(tbase, TK)], didxt)
    pltpu.sync_copy(rowst, acc_sh.at[didxt], add=True)
    plsc.subcore_barrier()

    @pl.when(cid == 0)
    def _():
        _writeback(acc_sh, acc0_hbm, sid)
    @pl.when(cid == 1)
    def _():
        _writeback(acc_sh, acc1_hbm, sid)


@functools.partial(
    pl.kernel,
    out_type=(jax.ShapeDtypeStruct((N, D), jnp.float32),
              jax.ShapeDtypeStruct((N, D), jnp.float32)),
    mesh=_mesh,
    scratch_types=[
        pltpu.VMEM_SHARED((N, D), jnp.float32),
        pltpu.VMEM((K,), jnp.int32),
        pltpu.VMEM((K,), jnp.int32),
        pltpu.VMEM((TK,), jnp.int32),
        pltpu.VMEM((K, D), jnp.float32),
        pltpu.VMEM((ZR, D), jnp.float32),
        pltpu.SemaphoreType.DMA,
        pltpu.SemaphoreType.DMA,
    ],
)
def _sc_deg(dst_hbm, deg0_hbm, deg1_hbm, deg_sh, didx0, didx1, didxt,
            ones_v, zbuf, sems0, sems1):
    cid = lax.axis_index("c")
    sid = lax.axis_index("s")
    wid = sid * NC + cid

    _zero_fill(zbuf, ZR, D)
    def fill_ones(i, _):
        for j in range(D // 16):
            ones_v[i, pl.ds(j * 16, 16)] = jnp.ones((16,), jnp.float32)
        return 0
    lax.fori_loop(0, K, fill_ones, 0)
    _zero_shared(deg_sh, zbuf, sid)
    plsc.subcore_barrier()

    ebase = wid * EPW

    def fire(c, didx, sems):
        pltpu.sync_copy(dst_hbm.at[pl.ds(ebase + c * K, K)], didx)
        pltpu.async_copy(ones_v, deg_sh.at[didx], sems, add=True)

    fire(0, didx0, sems0)
    fire(1, didx1, sems1)
    def pair(p, _):
        c = 2 * p
        pltpu.make_async_copy(ones_v, deg_sh.at[didx0], sems0).wait()
        @pl.when(c + 2 < CHUNKS)
        def _():
            fire(c + 2, didx0, sems0)
        pltpu.make_async_copy(ones_v, deg_sh.at[didx1], sems1).wait()
        @pl.when(c + 3 < CHUNKS)
        def _():
            fire(c + 3, didx1, sems1)
        return 0
    lax.fori_loop(0, CHUNKS // 2, pair, 0)

    tbase = ebase + CHUNKS * K
    pltpu.sync_copy(dst_hbm.at[pl.ds(tbase, TK)], didxt)
    pltpu.sync_copy(ones_v.at[pl.ds(0, TK)], deg_sh.at[didxt], add=True)
    plsc.subcore_barrier()

    @pl.when(cid == 0)
    def _():
        _writeback(deg_sh, deg0_hbm, sid)
    @pl.when(cid == 1)
    def _():
        _writeback(deg_sh, deg1_hbm, sid)


BN = 1000


def _dense_body(relu, h_ref, a0_ref, a1_ref, d0_ref, d1_ref, ws_ref, wn_ref,
                b_ref, o_ref):
    deg = jnp.maximum(d0_ref[:, 0:1] + d1_ref[:, 0:1], 1.0)
    hn = (a0_ref[...] + a1_ref[...]) / deg
    out = (
        jnp.dot(h_ref[...], ws_ref[...], preferred_element_type=jnp.float32,
                precision=lax.Precision.HIGHEST)
        + jnp.dot(hn, wn_ref[...], preferred_element_type=jnp.float32,
                  precision=lax.Precision.HIGHEST)
        + b_ref[...]
    )
    if relu:
        out = jnp.maximum(out, 0.0)
    o_ref[...] = out


def _dense(h, a0, a1, d0, d1, ws, wn, b, relu):
    return pl.pallas_call(
        functools.partial(_dense_body, relu),
        out_shape=jax.ShapeDtypeStruct((N, D), jnp.float32),
        grid=(N // BN,),
        in_specs=[
            pl.BlockSpec((BN, D), lambda i: (i, 0)),
            pl.BlockSpec((BN, D), lambda i: (i, 0)),
            pl.BlockSpec((BN, D), lambda i: (i, 0)),
            pl.BlockSpec((BN, D), lambda i: (i, 0)),
            pl.BlockSpec((BN, D), lambda i: (i, 0)),
            pl.BlockSpec((D, D), lambda i: (0, 0)),
            pl.BlockSpec((D, D), lambda i: (0, 0)),
            pl.BlockSpec((1, D), lambda i: (0, 0)),
        ],
        out_specs=pl.BlockSpec((BN, D), lambda i: (i, 0)),
    )(h, a0, a1, d0, d1, ws, wn, b)


def kernel(x, edge_index, W_self_1, W_neigh_1, b_1,
           W_self_2, W_neigh_2, b_2, W_self_3, W_neigh_3, b_3):
    src = edge_index[0].astype(jnp.int32)
    dst = edge_index[1].astype(jnp.int32)
    deg0, deg1 = _sc_deg(dst)
    h = x
    layers = [
        (W_self_1, W_neigh_1, b_1, True),
        (W_self_2, W_neigh_2, b_2, True),
        (W_self_3, W_neigh_3, b_3, False),
    ]
    for ws, wn, b, relu in layers:
        a0, a1 = _sc_agg(h, src, dst)
        h = _dense(h, a0, a1, deg0, deg1, ws, wn, b.reshape(1, D), relu)
    return h


# speedup vs baseline: 1.0630x; 1.0630x over previous
import functools

import jax
import jax.numpy as jnp
from jax import lax
from jax.experimental import pallas as pl
from jax.experimental.pallas import tpu as pltpu
from jax.experimental.pallas import tpu_sc as plsc

N = 10000
E = 320000
D = 128

NC = 2
NS = 16
NW = NC * NS
EPW = E // NW
K = 128
CHUNKS = EPW // K
TK = EPW - CHUNKS * K
NB = 2
GROUPS = CHUNKS // NB
RPT = 624
TAIL0 = NS * RPT
ZR = 40

_mesh = plsc.VectorSubcoreMesh(
    core_axis_name="c", subcore_axis_name="s", num_cores=NC, num_subcores=NS)


def _zero_fill(zbuf, rows, width):
    def row(i, _):
        for j in range(width // 16):
            zbuf[i, pl.ds(j * 16, 16)] = jnp.zeros((16,), jnp.float32)
        return 0
    lax.fori_loop(0, rows, row, 0)


def _zero_shared(sh, zbuf, sid):
    base = sid * RPT
    def z(i, _):
        pltpu.sync_copy(zbuf, sh.at[pl.ds(base + i * ZR, ZR)])
        return 0
    lax.fori_loop(0, 640 // ZR, z, 0)


def _writeback(src_sh, dst_hbm, sid):
    r0 = sid * RPT
    pltpu.sync_copy(src_sh.at[pl.ds(r0, RPT)], dst_hbm.at[pl.ds(r0, RPT)])
    @pl.when(sid == NS - 1)
    def _():
        pltpu.sync_copy(src_sh.at[pl.ds(TAIL0, N - TAIL0)],
                        dst_hbm.at[pl.ds(TAIL0, N - TAIL0)])


@functools.partial(
    pl.kernel,
    out_type=(jax.ShapeDtypeStruct((N, D), jnp.float32),
              jax.ShapeDtypeStruct((N, D), jnp.float32)),
    mesh=_mesh,
    scratch_types=(
        [pltpu.VMEM_SHARED((N, D), jnp.float32)]
        + [pltpu.VMEM((K,), jnp.int32)] * NB
        + [pltpu.VMEM((K,), jnp.int32)] * NB
        + [pltpu.VMEM((TK,), jnp.int32)] * 2
        + [pltpu.VMEM((K, D), jnp.float32)] * NB
        + [pltpu.VMEM((TK, D), jnp.float32)]
        + [pltpu.VMEM((ZR, D), jnp.float32)]
        + [pltpu.SemaphoreType.DMA] * (2 * NB)
    ),
)
def _sc_agg(x_hbm, src_hbm, dst_hbm, acc0_hbm, acc1_hbm, *s):
    acc_sh = s[0]
    sidx = s[1:1 + NB]
    didx = s[1 + NB:1 + 2 * NB]
    sidxt, didxt = s[1 + 2 * NB:3 + 2 * NB]
    rows = s[3 + 2 * NB:3 + 3 * NB]
    rowst = s[3 + 3 * NB]
    zbuf = s[4 + 3 * NB]
    semg = s[5 + 3 * NB:5 + 4 * NB]
    sems = s[5 + 4 * NB:5 + 5 * NB]
    cid = lax.axis_index("c")
    sid = lax.axis_index("s")
    wid = sid * NC + cid

    _zero_fill(zbuf, ZR, D)
    _zero_shared(acc_sh, zbuf, sid)
    plsc.subcore_barrier()

    ebase = wid * EPW

    def fire_gather(c, b):
        pltpu.sync_copy(src_hbm.at[pl.ds(ebase + c * K, K)], sidx[b])
        pltpu.async_copy(x_hbm.at[sidx[b]], rows[b], semg[b])
        pltpu.sync_copy(dst_hbm.at[pl.ds(ebase + c * K, K)], didx[b])

    for b in range(NB):
        fire_gather(b, b)

    def group(g, _):
        base = g * NB
        for b in range(NB):
            pltpu.make_async_copy(x_hbm.at[sidx[b]], rows[b], semg[b]).wait()
            pltpu.async_copy(rows[b], acc_sh.at[didx[b]], sems[b], add=True)
        for b in range(NB):
            c2 = base + NB + b
            pltpu.make_async_copy(rows[b], acc_sh.at[didx[b]], sems[b]).wait()
            @pl.when(c2 < CHUNKS)
            def _():
                fire_gather(c2, b)
        return 0
    lax.fori_loop(0, GROUPS, group, 0)

    tbase = ebase + CHUNKS * K
    pltpu.sync_copy(src_hbm.at[pl.ds(tbase, TK)], sidxt)
    pltpu.async_copy(x_hbm.at[sidxt], rowst, semg[0]).wait()
    pltpu.sync_copy(dst_hbm.at[pl.ds(tbase, TK)], didxt)
    pltpu.sync_copy(rowst, acc_sh.at[didxt], add=True)
    plsc.subcore_barrier()

    @pl.when(cid == 0)
    def _():
        _writeback(acc_sh, acc0_hbm, sid)
    @pl.when(cid == 1)
    def _():
        _writeback(acc_sh, acc1_hbm, sid)


@functools.partial(
    pl.kernel,
    out_type=(jax.ShapeDtypeStruct((N, D), jnp.float32),
              jax.ShapeDtypeStruct((N, D), jnp.float32)),
    mesh=_mesh,
    scratch_types=[
        pltpu.VMEM_SHARED((N, D), jnp.float32),
        pltpu.VMEM((K,), jnp.int32),
        pltpu.VMEM((K,), jnp.int32),
        pltpu.VMEM((TK,), jnp.int32),
        pltpu.VMEM((K, D), jnp.float32),
        pltpu.VMEM((ZR, D), jnp.float32),
        pltpu.SemaphoreType.DMA,
        pltpu.SemaphoreType.DMA,
    ],
)
def _sc_deg(dst_hbm, deg0_hbm, deg1_hbm, deg_sh, didx0, didx1, didxt,
            ones_v, zbuf, sems0, sems1):
    cid = lax.axis_index("c")
    sid = lax.axis_index("s")
    wid = sid * NC + cid

    _zero_fill(zbuf, ZR, D)
    def fill_ones(i, _):
        for j in range(D // 16):
            ones_v[i, pl.ds(j * 16, 16)] = jnp.ones((16,), jnp.float32)
        return 0
    lax.fori_loop(0, K, fill_ones, 0)
    _zero_shared(deg_sh, zbuf, sid)
    plsc.subcore_barrier()

    ebase = wid * EPW

    def fire(c, didx, sems):
        pltpu.sync_copy(dst_hbm.at[pl.ds(ebase + c * K, K)], didx)
        pltpu.async_copy(ones_v, deg_sh.at[didx], sems, add=True)

    fire(0, didx0, sems0)
    fire(1, didx1, sems1)
    def pair(p, _):
        c = 2 * p
        pltpu.make_async_copy(ones_v, deg_sh.at[didx0], sems0).wait()
        @pl.when(c + 2 < CHUNKS)
        def _():
            fire(c + 2, didx0, sems0)
        pltpu.make_async_copy(ones_v, deg_sh.at[didx1], sems1).wait()
        @pl.when(c + 3 < CHUNKS)
        def _():
            fire(c + 3, didx1, sems1)
        return 0
    lax.fori_loop(0, CHUNKS // 2, pair, 0)

    tbase = ebase + CHUNKS * K
    pltpu.sync_copy(dst_hbm.at[pl.ds(tbase, TK)], didxt)
    pltpu.sync_copy(ones_v.at[pl.ds(0, TK)], deg_sh.at[didxt], add=True)
    plsc.subcore_barrier()

    @pl.when(cid == 0)
    def _():
        _writeback(deg_sh, deg0_hbm, sid)
    @pl.when(cid == 1)
    def _():
        _writeback(deg_sh, deg1_hbm, sid)


BN = 1000


def _dense_body(relu, h_ref, a0_ref, a1_ref, d0_ref, d1_ref, ws_ref, wn_ref,
                b_ref, o_ref):
    deg = jnp.maximum(d0_ref[:, 0:1] + d1_ref[:, 0:1], 1.0)
    hn = (a0_ref[...] + a1_ref[...]) / deg
    out = (
        jnp.dot(h_ref[...], ws_ref[...], preferred_element_type=jnp.float32,
                precision=lax.Precision.HIGHEST)
        + jnp.dot(hn, wn_ref[...], preferred_element_type=jnp.float32,
                  precision=lax.Precision.HIGHEST)
        + b_ref[...]
    )
    if relu:
        out = jnp.maximum(out, 0.0)
    o_ref[...] = out


def _dense(h, a0, a1, d0, d1, ws, wn, b, relu):
    return pl.pallas_call(
        functools.partial(_dense_body, relu),
        out_shape=jax.ShapeDtypeStruct((N, D), jnp.float32),
        grid=(N // BN,),
        in_specs=[
            pl.BlockSpec((BN, D), lambda i: (i, 0)),
            pl.BlockSpec((BN, D), lambda i: (i, 0)),
            pl.BlockSpec((BN, D), lambda i: (i, 0)),
            pl.BlockSpec((BN, D), lambda i: (i, 0)),
            pl.BlockSpec((BN, D), lambda i: (i, 0)),
            pl.BlockSpec((D, D), lambda i: (0, 0)),
            pl.BlockSpec((D, D), lambda i: (0, 0)),
            pl.BlockSpec((1, D), lambda i: (0, 0)),
        ],
        out_specs=pl.BlockSpec((BN, D), lambda i: (i, 0)),
    )(h, a0, a1, d0, d1, ws, wn, b)


def kernel(x, edge_index, W_self_1, W_neigh_1, b_1,
           W_self_2, W_neigh_2, b_2, W_self_3, W_neigh_3, b_3):
    src = edge_index[0].astype(jnp.int32)
    dst = edge_index[1].astype(jnp.int32)
    deg0, deg1 = _sc_deg(dst)
    h = x
    layers = [
        (W_self_1, W_neigh_1, b_1, True),
        (W_self_2, W_neigh_2, b_2, True),
        (W_self_3, W_neigh_3, b_3, False),
    ]
    for ws, wn, b, relu in layers:
        a0, a1 = _sc_agg(h, src, dst)
        h = _dense(h, a0, a1, deg0, deg1, ws, wn, b.reshape(1, D), relu)
    return h
